# Initial kernel scaffold; baseline (speedup 1.0000x reference)
import jax
import jax.numpy as jnp
from jax.experimental import pallas as pl


def kernel(x, table):
    raise NotImplementedError("write your pallas kernel here")


# speedup vs baseline: 1.3643x; 1.3643x over previous
import functools
import math

import jax
import jax.numpy as jnp
import numpy as np
from jax import lax
from jax.experimental import pallas as pl
from jax.experimental.pallas import tpu as pltpu
from jax.experimental.pallas import tpu_sc as plsc

EMB = 32
VOCAB = 1000000
MAX_LEN = 5000
B = 4096
L = 200
SCALE = math.sqrt(EMB)

NC = 2
NS = 16
NW = NC * NS
TOT = B * L
PER_W = TOT // NW
CHUNK = 128
NCHUNK = PER_W // CHUNK
NB = 2
NGROUP = NCHUNK // NB


def _build_pe2():
    pos = np.arange(MAX_LEN, dtype=np.float32)[:, None]
    div = np.exp(np.arange(0, EMB, 2, dtype=np.float32) * (-math.log(10000.0) / EMB))
    pe = np.zeros((MAX_LEN, EMB), np.float32)
    pe[:, 0::2] = np.sin(pos * div)
    pe[:, 1::2] = np.cos(pos * div)
    pe = pe[:L]
    return np.concatenate([pe, pe], axis=0)


_PE2 = _build_pe2()

_mesh = plsc.VectorSubcoreMesh(core_axis_name="c", subcore_axis_name="s")


@functools.partial(
    pl.kernel,
    mesh=_mesh,
    compiler_params=pltpu.CompilerParams(use_tc_tiling_on_sc=False),
    out_type=jax.ShapeDtypeStruct((TOT, EMB), jnp.float32),
    scratch_types=[
        pltpu.VMEM((NCHUNK, CHUNK), jnp.int32),
        pltpu.VMEM((2 * L, EMB), jnp.float32),
        pltpu.VMEM((NB, CHUNK, EMB), jnp.float32),
        pltpu.VMEM((NB, CHUNK, EMB), jnp.float32),
        pltpu.SemaphoreType.DMA((NB,)),
        pltpu.SemaphoreType.DMA((NB,)),
    ],
)
def _emb_kernel(table, idx, pe2, out, idx_v, pe_v, gbuf, obuf, gsem, ssem):
    wid = lax.axis_index("s") * NC + lax.axis_index("c")
    base = wid * PER_W

    pltpu.sync_copy(idx.at[wid], idx_v)
    pltpu.sync_copy(pe2, pe_v)

    def start_gather(c, slot):
        pltpu.async_copy(table.at[idx_v.at[c]], gbuf.at[slot], gsem.at[slot])

    def wait_gather(c, slot):
        pltpu.make_async_copy(
            table.at[idx_v.at[c]], gbuf.at[slot], gsem.at[slot]
        ).wait()

    def start_scatter(c, slot):
        pltpu.async_copy(
            obuf.at[slot], out.at[pl.ds(base + c * CHUNK, CHUNK)], ssem.at[slot]
        )

    def wait_scatter(c, slot):
        pltpu.make_async_copy(
            obuf.at[slot], out.at[pl.ds(base + c * CHUNK, CHUNK)], ssem.at[slot]
        ).wait()

    def compute(c, slot):
        p0 = lax.rem(c * CHUNK, L)

        def body(r4, _):
            for j in range(4):
                r = r4 * 4 + j
                p = p0 + r
                g0 = gbuf[slot, r, pl.ds(0, 16)]
                g1 = gbuf[slot, r, pl.ds(16, 16)]
                e0 = pe_v[p, pl.ds(0, 16)]
                e1 = pe_v[p, pl.ds(16, 16)]
                obuf[slot, r, pl.ds(0, 16)] = g0 * SCALE + e0
                obuf[slot, r, pl.ds(16, 16)] = g1 * SCALE + e1
            return _

        lax.fori_loop(0, CHUNK // 4, body, None)

    for b in range(NB):
        start_gather(b, b)

    for b in range(NB):
        wait_gather(b, b)
        compute(b, b)
        start_scatter(b, b)
        start_gather(b + NB, b)

    def group(g, _):
        for b in range(NB):
            c = g * NB + b
            wait_gather(c, b)
            wait_scatter(c - NB, b)
            compute(c, b)
            start_scatter(c, b)
            start_gather(c + NB, b)
        return _

    lax.fori_loop(1, NGROUP - 1, group, None)

    for b in range(NB):
        c = (NGROUP - 1) * NB + b
        wait_gather(c, b)
        wait_scatter(c - NB, b)
        compute(c, b)
        start_scatter(c, b)

    for b in range(NB):
        wait_scatter((NGROUP - 1) * NB + b, b)


def kernel(x, table):
    idx = x.reshape(NW, NCHUNK, CHUNK).astype(jnp.int32)
    out = _emb_kernel(table, idx, jnp.asarray(_PE2))
    return out.reshape(B, L, EMB)


# speedup vs baseline: 1.3878x; 1.0172x over previous
import functools
import math

import jax
import jax.numpy as jnp
import numpy as np
from jax import lax
from jax.experimental import pallas as pl
from jax.experimental.pallas import tpu as pltpu
from jax.experimental.pallas import tpu_sc as plsc

EMB = 32
VOCAB = 1000000
MAX_LEN = 5000
B = 4096
L = 200
SCALE = math.sqrt(EMB)

NC = 2
NS = 16
NW = NC * NS
ROWS_W = B // NW
CHUNK = 104
OFF1 = 96
OUT_COLS = 128
OUT_ROWS = B * L * EMB // OUT_COLS
ROWS_PER_CHUNK = CHUNK * EMB // OUT_COLS
ROW_OUT = L * EMB // OUT_COLS
W_OUT_ROWS = ROWS_W * ROW_OUT


def _build_pe():
    pos = np.arange(MAX_LEN, dtype=np.float32)[:, None]
    div = np.exp(np.arange(0, EMB, 2, dtype=np.float32) * (-math.log(10000.0) / EMB))
    pe = np.zeros((MAX_LEN, EMB), np.float32)
    pe[:, 0::2] = np.sin(pos * div)
    pe[:, 1::2] = np.cos(pos * div)
    return pe[:L]


_PE = _build_pe()

_mesh = plsc.VectorSubcoreMesh(core_axis_name="c", subcore_axis_name="s")


@functools.partial(
    pl.kernel,
    mesh=_mesh,
    compiler_params=pltpu.CompilerParams(use_tc_tiling_on_sc=False),
    out_type=jax.ShapeDtypeStruct((OUT_ROWS, OUT_COLS), jnp.float32),
    scratch_types=[
        pltpu.VMEM((ROWS_W, L), jnp.int32),
        pltpu.VMEM((L, EMB), jnp.float32),
        pltpu.VMEM((2, CHUNK, EMB), jnp.float32),
        pltpu.VMEM((2, ROWS_PER_CHUNK, OUT_COLS), jnp.float32),
        pltpu.SemaphoreType.DMA((2,)),
        pltpu.SemaphoreType.DMA((2,)),
    ],
)
def _emb_kernel(table, x, pe, out, idx_v, pe_v, gbuf, obuf, gsem, ssem):
    wid = lax.axis_index("s") * NC + lax.axis_index("c")
    row0 = wid * ROWS_W
    obase = wid * W_OUT_ROWS

    pltpu.sync_copy(x.at[pl.ds(row0, ROWS_W)], idx_v)
    pltpu.sync_copy(pe, pe_v)

    def gather_desc(g, b):
        return pltpu.make_async_copy(
            table.at[idx_v.at[g, pl.ds(b * OFF1, CHUNK)]],
            gbuf.at[b],
            gsem.at[b],
        )

    def scatter_desc(g, b):
        orow = obase + g * ROW_OUT + b * (OFF1 * EMB // OUT_COLS)
        return pltpu.make_async_copy(
            obuf.at[b],
            out.at[pl.ds(orow, ROWS_PER_CHUNK)],
            ssem.at[b],
        )

    def compute(b):
        def body(it, carry):
            for j in range(4):
                rr = it * 4 + j
                p = b * OFF1 + rr
                g0 = gbuf[b, rr, pl.ds(0, 16)]
                g1 = gbuf[b, rr, pl.ds(16, 16)]
                e0 = pe_v[p, pl.ds(0, 16)]
                e1 = pe_v[p, pl.ds(16, 16)]
                obuf[b, it, pl.ds(j * 32, 16)] = g0 * SCALE + e0
                obuf[b, it, pl.ds(j * 32 + 16, 16)] = g1 * SCALE + e1
            return carry

        lax.fori_loop(0, ROWS_PER_CHUNK, body, None)

    for b in range(2):
        gather_desc(0, b).start()

    for b in range(2):
        gather_desc(0, b).wait()
        compute(b)
        scatter_desc(0, b).start()
        gather_desc(1, b).start()

    def row_step(g, carry):
        for b in range(2):
            gather_desc(g, b).wait()
            scatter_desc(g - 1, b).wait()
            compute(b)
            scatter_desc(g, b).start()
            gather_desc(g + 1, b).start()
        return carry

    lax.fori_loop(1, ROWS_W - 1, row_step, None)

    g_last = ROWS_W - 1
    for b in range(2):
        gather_desc(g_last, b).wait()
        scatter_desc(g_last - 1, b).wait()
        compute(b)
        scatter_desc(g_last, b).start()

    for b in range(2):
        scatter_desc(g_last, b).wait()


def kernel(x, table):
    out = _emb_kernel(table, x.astype(jnp.int32), jnp.asarray(_PE))
    return out.reshape(B, L, EMB)
